# baseline (speedup 1.0000x reference)
import functools

import jax
import jax.numpy as jnp
from jax import lax
from jax.experimental import pallas as pl
from jax.experimental.pallas import tpu as pltpu
from jax.experimental.pallas import tpu_sc as plsc

N = 10000
E = 320000
D = 128
H = D // 2

NC = 2
NS = 16
L = 16

CHUNK = 64
NBUF = 8
TCH = 320
SEG = 80
NSEG = TCH // SEG
NCHUNKS = NS * TCH
EP = NCHUNKS * CHUNK
STRIPE = 624

_SPLAT_DNUMS = jax.lax.GatherDimensionNumbers(
    offset_dims=(), collapsed_slice_dims=(0,), start_index_map=(0,))


def _sc_segment_sum(in_lo, in_hi, src, dst, w):
  mesh = plsc.VectorSubcoreMesh(core_axis_name="c", subcore_axis_name="s")

  @functools.partial(
      pl.kernel,
      out_type=jax.ShapeDtypeStruct((NC, N, H), jnp.float32),
      mesh=mesh,
      compiler_params=pltpu.CompilerParams(use_tc_tiling_on_sc=False),
      scratch_types=[
          pltpu.VMEM((SEG, CHUNK), jnp.int32),
          pltpu.VMEM((SEG, CHUNK), jnp.int32),
          pltpu.VMEM((SEG, CHUNK), jnp.float32),
          pltpu.VMEM((NBUF, CHUNK, H), jnp.float32),
          pltpu.VMEM_SHARED((N, H), jnp.float32),
          pltpu.SemaphoreType.DMA,
          pltpu.SemaphoreType.DMA,
          pltpu.SemaphoreType.DMA,
          pltpu.SemaphoreType.DMA,
          pltpu.SemaphoreType.DMA,
          pltpu.SemaphoreType.DMA,
          pltpu.SemaphoreType.DMA,
          pltpu.SemaphoreType.DMA,
          pltpu.SemaphoreType.DMA,
          pltpu.SemaphoreType.DMA,
          pltpu.SemaphoreType.DMA,
          pltpu.SemaphoreType.DMA,
          pltpu.SemaphoreType.DMA,
          pltpu.SemaphoreType.DMA,
          pltpu.SemaphoreType.DMA,
          pltpu.SemaphoreType.DMA,
          pltpu.SemaphoreType.DMA,
      ],
  )
  def k(lo_hbm, hi_hbm, src_hbm, dst_hbm, w_hbm, out_hbm,
        src_v, dst_v, w_v, rows_v, acc_sh, sem_p,
        sg0, sg1, sg2, sg3, sg4, sg5, sg6, sg7,
        ss0, ss1, ss2, ss3, ss4, ss5, ss6, ss7):
    sgs = [sg0, sg1, sg2, sg3, sg4, sg5, sg6, sg7]
    sss = [ss0, ss1, ss2, ss3, ss4, ss5, ss6, ss7]
    cid = lax.axis_index("c")
    sid = lax.axis_index("s")
    ch0 = sid * TCH

    def load_tables(seg_base):
      d_src = pltpu.async_copy(src_hbm.at[pl.ds(seg_base, SEG)], src_v, sem_p)
      d_dst = pltpu.async_copy(dst_hbm.at[pl.ds(seg_base, SEG)], dst_v, sem_p)
      d_w = pltpu.async_copy(w_hbm.at[pl.ds(seg_base, SEG)], w_v, sem_p)
      return d_src, d_dst, d_w

    def wait_tables(descs):
      for d in descs:
        d.wait()

    descs0 = load_tables(ch0)

    zeros16 = jnp.zeros((L,), jnp.float32)

    def zrow(i, c):
      for j in range(H // L):
        rows_v[0, i, pl.ds(j * L, L)] = zeros16
      return c

    lax.fori_loop(0, CHUNK, zrow, 0, unroll=False)
    base_row = sid * STRIPE

    def zcopy(i, c):
      pltpu.sync_copy(rows_v.at[0],
                      acc_sh.at[pl.ds(base_row + i * CHUNK, CHUNK)])
      return c

    @pl.when(sid < NS - 1)
    def _():
      lax.fori_loop(0, 9, zcopy, 0, unroll=False)
      pltpu.sync_copy(rows_v.at[0, pl.ds(0, STRIPE - 576)],
                      acc_sh.at[pl.ds(base_row + 576, STRIPE - 576)])

    @pl.when(sid == NS - 1)
    def _():
      lax.fori_loop(0, 10, zcopy, 0, unroll=False)

    wait_tables(descs0)
    plsc.subcore_barrier()

    def gather(t, b):
      @pl.when(cid == 0)
      def _():
        pltpu.async_copy(lo_hbm.at[src_v.at[t]], rows_v.at[b], sgs[b])

      @pl.when(cid == 1)
      def _():
        pltpu.async_copy(hi_hbm.at[src_v.at[t]], rows_v.at[b], sgs[b])

    def scale(t, b):
      def scale16(a, c):
        w16 = w_v[t, pl.ds(a * L, L)]
        lane = lax.iota(jnp.int32, L)
        for bb in range(L):
          bidx = ((lane * 0) + bb).reshape(L, 1)
          wsplat = lax.gather(
              w16, bidx, _SPLAT_DNUMS, slice_sizes=(1,),
              mode=lax.GatherScatterMode.PROMISE_IN_BOUNDS)
          r = a * L + bb
          for j in range(H // L):
            sl = pl.ds(j * L, L)
            rows_v[b, r, sl] = rows_v[b, r, sl] * wsplat
        return c

      lax.fori_loop(0, CHUNK // L, scale16, 0, unroll=False)

    def run_segment():
      for b in range(NBUF // 2):
        gather(b, b)

      def step(i, carry):
        for ph in range(NBUF):
          t = i * NBUF + ph
          b_r = (ph + NBUF // 2) % NBUF

          pltpu.make_async_copy(lo_hbm.at[src_v.at[t]], rows_v.at[ph],
                                sgs[ph]).wait()

          @pl.when(t >= NBUF // 2)
          def _():
            pltpu.make_async_copy(rows_v.at[b_r], acc_sh.at[dst_v.at[0]],
                                  sss[b_r]).wait()

          @pl.when(t + NBUF // 2 < SEG)
          def _():
            gather(t + NBUF // 2, b_r)

          scale(t, ph)
          pltpu.async_copy(rows_v.at[ph], acc_sh.at[dst_v.at[t]], sss[ph],
                           add=True)

        return carry

      lax.fori_loop(0, SEG // NBUF, step, 0, unroll=False)

      for d in range(NBUF // 2):
        b = (SEG - NBUF // 2 + d) % NBUF
        pltpu.make_async_copy(rows_v.at[b], acc_sh.at[dst_v.at[0]],
                              sss[b]).wait()

    run_segment()
    for s in range(1, NSEG):
      wait_tables(load_tables(ch0 + s * SEG))
      run_segment()

    plsc.subcore_barrier()

    @pl.when(sid < NS - 1)
    def _():
      pltpu.sync_copy(acc_sh.at[pl.ds(base_row, STRIPE)],
                      out_hbm.at[cid, pl.ds(base_row, STRIPE)])

    @pl.when(sid == NS - 1)
    def _():
      last = (NS - 1) * STRIPE
      pltpu.sync_copy(acc_sh.at[pl.ds(last, N - last)],
                      out_hbm.at[cid, pl.ds(last, N - last)])

  return k(in_lo, in_hi, src, dst, w)


BLK = 1000


def _tc_body(x_ref, p_ref, w_ref, o_ref):
  w = w_ref[...]
  o_ref[:, :D] = jnp.dot(x_ref[...], w, preferred_element_type=jnp.float32)
  nb = jnp.concatenate([p_ref[0], p_ref[1]], axis=1)
  o_ref[:, D:] = jnp.dot(nb, w, preferred_element_type=jnp.float32)


def _tc_transform(input_, halves, W):
  return pl.pallas_call(
      _tc_body,
      grid=(N // BLK,),
      in_specs=[
          pl.BlockSpec((BLK, D), lambda i: (i, 0)),
          pl.BlockSpec((NC, BLK, H), lambda i: (0, i, 0)),
          pl.BlockSpec((D, D), lambda i: (0, 0)),
      ],
      out_specs=pl.BlockSpec((BLK, 2 * D), lambda i: (i, 0)),
      out_shape=jax.ShapeDtypeStruct((N, 2 * D), jnp.float32),
  )(input_, halves, W)


@jax.jit
def kernel(input_, edge_index, edge_weight, W):
  pad = EP - E
  pad_idx = jnp.arange(pad, dtype=jnp.int32) % N
  src = jnp.concatenate([edge_index[1].astype(jnp.int32), pad_idx])
  dst = jnp.concatenate([edge_index[0].astype(jnp.int32), pad_idx])
  w = jnp.concatenate([edge_weight, jnp.zeros((pad,), jnp.float32)])
  halves = _sc_segment_sum(input_[:, :H], input_[:, H:],
                           src.reshape(NCHUNKS, CHUNK),
                           dst.reshape(NCHUNKS, CHUNK),
                           w.reshape(NCHUNKS, CHUNK))
  return _tc_transform(input_, halves, W)


# speedup vs baseline: 1.7094x; 1.7094x over previous
import functools

import jax
import jax.numpy as jnp
from jax import lax
from jax.experimental import pallas as pl
from jax.experimental.pallas import tpu as pltpu
from jax.experimental.pallas import tpu_sc as plsc

N = 10000
E = 320000
D = 128

NC = 2
NS = 16
L = 16
NW = NC * NS

CHUNK = 64
NBUF = 4
TCH = 160
SEG = 40
NSEG = TCH // SEG
NCHUNKS = NW * TCH
EP = NCHUNKS * CHUNK
STRIPE = 624

_SPLAT_DNUMS = jax.lax.GatherDimensionNumbers(
    offset_dims=(), collapsed_slice_dims=(0,), start_index_map=(0,))


def _sc_segment_sum(input_, src, dst, w):
  mesh = plsc.VectorSubcoreMesh(core_axis_name="c", subcore_axis_name="s")

  @functools.partial(
      pl.kernel,
      out_type=jax.ShapeDtypeStruct((NC, N, D), jnp.float32),
      mesh=mesh,
      scratch_types=[
          pltpu.VMEM((SEG, CHUNK), jnp.int32),
          pltpu.VMEM((SEG, CHUNK), jnp.int32),
          pltpu.VMEM((SEG, CHUNK), jnp.float32),
          pltpu.VMEM((NBUF, CHUNK, D), jnp.float32),
          pltpu.VMEM_SHARED((N, D), jnp.float32),
          pltpu.SemaphoreType.DMA,
          pltpu.SemaphoreType.DMA,
          pltpu.SemaphoreType.DMA,
          pltpu.SemaphoreType.DMA,
          pltpu.SemaphoreType.DMA,
          pltpu.SemaphoreType.DMA,
          pltpu.SemaphoreType.DMA,
          pltpu.SemaphoreType.DMA,
          pltpu.SemaphoreType.DMA,
      ],
  )
  def k(input_hbm, src_hbm, dst_hbm, w_hbm, out_hbm,
        src_v, dst_v, w_v, rows_v, acc_sh, sem_p,
        sg0, sg1, sg2, sg3, ss0, ss1, ss2, ss3):
    sgs = [sg0, sg1, sg2, sg3]
    sss = [ss0, ss1, ss2, ss3]
    cid = lax.axis_index("c")
    sid = lax.axis_index("s")
    wid = cid * NS + sid
    ch0 = wid * TCH

    def load_tables(seg_base):
      d_src = pltpu.async_copy(src_hbm.at[pl.ds(seg_base, SEG)], src_v, sem_p)
      d_dst = pltpu.async_copy(dst_hbm.at[pl.ds(seg_base, SEG)], dst_v, sem_p)
      d_w = pltpu.async_copy(w_hbm.at[pl.ds(seg_base, SEG)], w_v, sem_p)
      return d_src, d_dst, d_w

    def wait_tables(descs):
      for d in descs:
        d.wait()

    descs0 = load_tables(ch0)

    zeros16 = jnp.zeros((L,), jnp.float32)

    def zrow(i, c):
      for j in range(D // L):
        rows_v[0, i, pl.ds(j * L, L)] = zeros16
      return c

    lax.fori_loop(0, CHUNK, zrow, 0, unroll=False)
    base_row = sid * STRIPE

    def zcopy(i, c):
      pltpu.sync_copy(rows_v.at[0],
                      acc_sh.at[pl.ds(base_row + i * CHUNK, CHUNK)])
      return c

    @pl.when(sid < NS - 1)
    def _():
      lax.fori_loop(0, 9, zcopy, 0, unroll=False)
      pltpu.sync_copy(rows_v.at[0, pl.ds(0, STRIPE - 576)],
                      acc_sh.at[pl.ds(base_row + 576, STRIPE - 576)])

    @pl.when(sid == NS - 1)
    def _():
      lax.fori_loop(0, 10, zcopy, 0, unroll=False)

    wait_tables(descs0)
    plsc.subcore_barrier()

    def scale(t, b):
      def scale16(a, c):
        w16 = w_v[t, pl.ds(a * L, L)]
        lane = lax.iota(jnp.int32, L)
        for bb in range(L):
          bidx = ((lane * 0) + bb).reshape(L, 1)
          wsplat = lax.gather(
              w16, bidx, _SPLAT_DNUMS, slice_sizes=(1,),
              mode=lax.GatherScatterMode.PROMISE_IN_BOUNDS)
          r = a * L + bb
          for j in range(D // L):
            sl = pl.ds(j * L, L)
            rows_v[b, r, sl] = rows_v[b, r, sl] * wsplat
        return c

      lax.fori_loop(0, CHUNK // L, scale16, 0, unroll=False)

    def run_segment():
      for b in range(NBUF - 1):
        pltpu.async_copy(input_hbm.at[src_v.at[b]], rows_v.at[b], sgs[b])

      def step(i, carry):
        for ph in range(NBUF):
          t = i * NBUF + ph
          b_n = (ph + NBUF - 1) % NBUF

          pltpu.make_async_copy(input_hbm.at[src_v.at[t]], rows_v.at[ph],
                                sgs[ph]).wait()
          scale(t, ph)
          pltpu.async_copy(rows_v.at[ph], acc_sh.at[dst_v.at[t]], sss[ph],
                           add=True)

          @pl.when(t >= 1)
          def _():
            pltpu.make_async_copy(rows_v.at[b_n], acc_sh.at[dst_v.at[0]],
                                  sss[b_n]).wait()

          @pl.when(t + NBUF - 1 < SEG)
          def _():
            pltpu.async_copy(input_hbm.at[src_v.at[t + NBUF - 1]],
                             rows_v.at[b_n], sgs[b_n])

        return carry

      lax.fori_loop(0, SEG // NBUF, step, 0, unroll=False)

      last_b = (SEG - 1) % NBUF
      pltpu.make_async_copy(rows_v.at[last_b], acc_sh.at[dst_v.at[0]],
                            sss[last_b]).wait()

    run_segment()
    for s in range(1, NSEG):
      wait_tables(load_tables(ch0 + s * SEG))
      run_segment()

    plsc.subcore_barrier()

    @pl.when(sid < NS - 1)
    def _():
      pltpu.sync_copy(acc_sh.at[pl.ds(base_row, STRIPE)],
                      out_hbm.at[cid, pl.ds(base_row, STRIPE)])

    @pl.when(sid == NS - 1)
    def _():
      last = (NS - 1) * STRIPE
      pltpu.sync_copy(acc_sh.at[pl.ds(last, N - last)],
                      out_hbm.at[cid, pl.ds(last, N - last)])

  return k(input_, src, dst, w)


BLK = 1000


def _tc_body(x_ref, p_ref, w_ref, o_ref):
  w = w_ref[...]
  o_ref[:, :D] = jnp.dot(x_ref[...], w, preferred_element_type=jnp.float32)
  nb = p_ref[0] + p_ref[1]
  o_ref[:, D:] = jnp.dot(nb, w, preferred_element_type=jnp.float32)


def _tc_transform(input_, partials, W):
  return pl.pallas_call(
      _tc_body,
      grid=(N // BLK,),
      in_specs=[
          pl.BlockSpec((BLK, D), lambda i: (i, 0)),
          pl.BlockSpec((NC, BLK, D), lambda i: (0, i, 0)),
          pl.BlockSpec((D, D), lambda i: (0, 0)),
      ],
      out_specs=pl.BlockSpec((BLK, 2 * D), lambda i: (i, 0)),
      out_shape=jax.ShapeDtypeStruct((N, 2 * D), jnp.float32),
  )(input_, partials, W)


@jax.jit
def kernel(input_, edge_index, edge_weight, W):
  pad = EP - E
  pad_idx = jnp.arange(pad, dtype=jnp.int32) % N
  src = jnp.concatenate([edge_index[1].astype(jnp.int32), pad_idx])
  dst = jnp.concatenate([edge_index[0].astype(jnp.int32), pad_idx])
  w = jnp.concatenate([edge_weight, jnp.zeros((pad,), jnp.float32)])
  partials = _sc_segment_sum(input_, src.reshape(NCHUNKS, CHUNK),
                             dst.reshape(NCHUNKS, CHUNK),
                             w.reshape(NCHUNKS, CHUNK))
  return _tc_transform(input_, partials, W)
